# Initial kernel scaffold; baseline (speedup 1.0000x reference)
import jax
import jax.numpy as jnp
from jax.experimental import pallas as pl


def kernel(x, u, w, b):
    raise NotImplementedError("write your pallas kernel here")


# speedup vs baseline: 1.0248x; 1.0248x over previous
import functools

import jax
import jax.numpy as jnp
import numpy as np
from jax.experimental import pallas as pl
from jax.experimental.pallas import tpu as pltpu

_PACK = 256
_SUBLANES = 8


def _packed_kernel(scal_ref, x_ref, bsel_ref, urep_ref, idx_ref, out_ref,
                   ld_ref, *, p, d):
    b = scal_ref[0]
    wtu = scal_ref[1]

    xf = x_ref[...].astype(jnp.float32)

    lin_b = jnp.dot(xf, bsel_ref[...],
                    preferred_element_type=jnp.float32) + b
    h_b = jnp.tanh(lin_b)

    out_ref[...] = (xf + h_b * urep_ref[...]).astype(out_ref.dtype)

    tm = h_b.shape[0]
    half = p // 2
    idx = jnp.broadcast_to(idx_ref[...], (tm, half))
    h_lo = jnp.take_along_axis(h_b[:, :128], idx, axis=1)
    h_hi = jnp.take_along_axis(h_b[:, 128:], idx, axis=1)
    h_k = jnp.concatenate([h_lo, h_hi], axis=1)
    ld_ref[...] = jnp.log(jnp.abs(1.0 + (1.0 - h_k * h_k) * wtu))


def _rowwise_kernel(scal_ref, x_ref, w_ref, u_ref, out_ref, ld_ref):
    b = scal_ref[0]
    wtu = scal_ref[1]

    xf = x_ref[...].astype(jnp.float32)
    lin = jnp.sum(xf * w_ref[...], axis=-1, keepdims=True) + b
    h = jnp.tanh(lin)
    out_ref[...] = (xf + h * u_ref[...]).astype(out_ref.dtype)
    ld_ref[...] = jnp.log(jnp.abs(1.0 + (1.0 - h * h) * wtu))


def _u_hat_scalars(u, w, b, d):
    wf = w.astype(jnp.float32).reshape(1, d)
    uf = u.astype(jnp.float32).reshape(1, d)
    wtu = jnp.sum(wf * uf)
    m_wtu = -1.0 + jnp.log1p(jnp.exp(wtu))
    u_hat = jnp.where(wtu < 1.0, uf + (m_wtu - wtu) * wf / jnp.sum(wf * wf), uf)
    wtu_hat = jnp.sum(wf * u_hat)
    scalars = jnp.stack([jnp.reshape(b, ()).astype(jnp.float32), wtu_hat])
    return wf, u_hat, scalars


def kernel(x, u, w, b):
    N, d = x.shape
    wf, u_hat, scalars = _u_hat_scalars(u, w, b, d)
    isz = jnp.dtype(x.dtype).itemsize

    packable = (d <= 128) and (128 % d == 0) and (N % (_PACK // d) == 0)

    if packable:
        p = _PACK // d
        M = N // p
        x_packed = x.reshape(M, _PACK)

        lane = np.arange(_PACK)
        seg = (lane[:, None] // d == lane[None, :] // d).astype(np.float32)
        w_rep = jnp.tile(wf.reshape(d), p)
        u_rep = jnp.tile(u_hat.reshape(d), p).reshape(1, _PACK)
        bsel = seg * w_rep[:, None]
        idx = jnp.asarray(lane[:128:d].reshape(1, p // 2), dtype=jnp.int32)

        tile = 8192
        while tile > _SUBLANES and M % tile != 0:
            tile //= 2
        if M % tile != 0:
            tile = M
        grid = (M // tile,)
        block_bytes = tile * _PACK * (isz + 4) + tile * p * 4
        vmem_limit = int(min(60000 * 1024,
                             2 * block_bytes + (24 << 20)))

        out_p, ld_p = pl.pallas_call(
            functools.partial(_packed_kernel, p=p, d=d),
            out_shape=(
                jax.ShapeDtypeStruct((M, _PACK), x.dtype),
                jax.ShapeDtypeStruct((M, p), jnp.float32),
            ),
            grid_spec=pltpu.PrefetchScalarGridSpec(
                num_scalar_prefetch=0,
                grid=grid,
                in_specs=[
                    pl.BlockSpec(memory_space=pltpu.MemorySpace.SMEM),
                    pl.BlockSpec((tile, _PACK), lambda i: (i, 0)),
                    pl.BlockSpec((_PACK, _PACK), lambda i: (0, 0)),
                    pl.BlockSpec((1, _PACK), lambda i: (0, 0)),
                    pl.BlockSpec((1, p // 2), lambda i: (0, 0)),
                ],
                out_specs=[
                    pl.BlockSpec((tile, _PACK), lambda i: (i, 0)),
                    pl.BlockSpec((tile, p), lambda i: (i, 0)),
                ],
            ),
            compiler_params=pltpu.CompilerParams(
                dimension_semantics=("parallel",),
                vmem_limit_bytes=vmem_limit),
        )(scalars, x_packed, bsel, u_rep, idx)

        return out_p.reshape(N, d), ld_p.reshape(N, 1)

    tile = min(N, 4096)
    tile = max(_SUBLANES, (tile // _SUBLANES) * _SUBLANES)
    grid = (pl.cdiv(N, tile),)
    out, ld = pl.pallas_call(
        _rowwise_kernel,
        out_shape=(
            jax.ShapeDtypeStruct((N, d), x.dtype),
            jax.ShapeDtypeStruct((N, 1), jnp.float32),
        ),
        grid_spec=pltpu.PrefetchScalarGridSpec(
            num_scalar_prefetch=0,
            grid=grid,
            in_specs=[
                pl.BlockSpec(memory_space=pltpu.MemorySpace.SMEM),
                pl.BlockSpec((tile, d), lambda i: (i, 0)),
                pl.BlockSpec((1, d), lambda i: (0, 0)),
                pl.BlockSpec((1, d), lambda i: (0, 0)),
            ],
            out_specs=[
                pl.BlockSpec((tile, d), lambda i: (i, 0)),
                pl.BlockSpec((tile, 1), lambda i: (i, 0)),
            ],
        ),
        compiler_params=pltpu.CompilerParams(
            dimension_semantics=("parallel",),
            vmem_limit_bytes=48 * 1024 * 1024),
    )(scalars, x, wf, u_hat)
    return out, ld


# speedup vs baseline: 11.8352x; 11.5488x over previous
import jax
import jax.numpy as jnp
from jax.experimental import pallas as pl
from jax.experimental.pallas import tpu as pltpu

_LANES = 128
_SUBLANES = 8


def _colwise_kernel(scal_ref, xt_ref, w_ref, u_ref, out_ref, ld_ref):
    b = scal_ref[0]
    wtu = scal_ref[1]

    xf = xt_ref[...].astype(jnp.float32)
    lin = jnp.sum(xf * w_ref[...], axis=0, keepdims=True) + b
    h = jnp.tanh(lin)
    out_ref[...] = (xf + u_ref[...] * h).astype(out_ref.dtype)
    ld_ref[...] = jnp.log(jnp.abs(1.0 + (1.0 - h * h) * wtu))


def _rowwise_kernel(scal_ref, x_ref, w_ref, u_ref, out_ref, ld_ref):
    b = scal_ref[0]
    wtu = scal_ref[1]

    xf = x_ref[...].astype(jnp.float32)
    lin = jnp.sum(xf * w_ref[...], axis=-1, keepdims=True) + b
    h = jnp.tanh(lin)
    out_ref[...] = (xf + h * u_ref[...]).astype(out_ref.dtype)
    ld_ref[...] = jnp.log(jnp.abs(1.0 + (1.0 - h * h) * wtu))


def _u_hat_scalars(u, w, b, d):
    wf = w.astype(jnp.float32).reshape(1, d)
    uf = u.astype(jnp.float32).reshape(1, d)
    wtu = jnp.sum(wf * uf)
    m_wtu = -1.0 + jnp.log1p(jnp.exp(wtu))
    u_hat = jnp.where(wtu < 1.0, uf + (m_wtu - wtu) * wf / jnp.sum(wf * wf), uf)
    wtu_hat = jnp.sum(wf * u_hat)
    scalars = jnp.stack([jnp.reshape(b, ()).astype(jnp.float32), wtu_hat])
    return wf, u_hat, scalars


def kernel(x, u, w, b):
    N, d = x.shape
    wf, u_hat, scalars = _u_hat_scalars(u, w, b, d)

    if d % _SUBLANES == 0 and N % _LANES == 0:
        xt = x.T
        w_col = wf.reshape(d, 1)
        u_col = u_hat.reshape(d, 1)

        tl = 65536
        while tl > _LANES and N % tl != 0:
            tl //= 2
        grid = (N // tl,)
        block_bytes = d * tl * 8 + tl * 4
        vmem_limit = int(min(60000 * 1024, 2 * block_bytes + (16 << 20)))

        out_t, ld_row = pl.pallas_call(
            _colwise_kernel,
            out_shape=(
                jax.ShapeDtypeStruct((d, N), x.dtype),
                jax.ShapeDtypeStruct((1, N), jnp.float32),
            ),
            grid_spec=pltpu.PrefetchScalarGridSpec(
                num_scalar_prefetch=0,
                grid=grid,
                in_specs=[
                    pl.BlockSpec(memory_space=pltpu.MemorySpace.SMEM),
                    pl.BlockSpec((d, tl), lambda i: (0, i)),
                    pl.BlockSpec((d, 1), lambda i: (0, 0)),
                    pl.BlockSpec((d, 1), lambda i: (0, 0)),
                ],
                out_specs=[
                    pl.BlockSpec((d, tl), lambda i: (0, i)),
                    pl.BlockSpec((1, tl), lambda i: (0, i)),
                ],
            ),
            compiler_params=pltpu.CompilerParams(
                dimension_semantics=("parallel",),
                vmem_limit_bytes=vmem_limit),
        )(scalars, xt, w_col, u_col)

        return out_t.T, ld_row.reshape(N, 1)

    tile = min(N, 4096)
    tile = max(_SUBLANES, (tile // _SUBLANES) * _SUBLANES)
    grid = (pl.cdiv(N, tile),)
    out, ld = pl.pallas_call(
        _rowwise_kernel,
        out_shape=(
            jax.ShapeDtypeStruct((N, d), x.dtype),
            jax.ShapeDtypeStruct((N, 1), jnp.float32),
        ),
        grid_spec=pltpu.PrefetchScalarGridSpec(
            num_scalar_prefetch=0,
            grid=grid,
            in_specs=[
                pl.BlockSpec(memory_space=pltpu.MemorySpace.SMEM),
                pl.BlockSpec((tile, d), lambda i: (i, 0)),
                pl.BlockSpec((1, d), lambda i: (0, 0)),
                pl.BlockSpec((1, d), lambda i: (0, 0)),
            ],
            out_specs=[
                pl.BlockSpec((tile, d), lambda i: (i, 0)),
                pl.BlockSpec((tile, 1), lambda i: (i, 0)),
            ],
        ),
        compiler_params=pltpu.CompilerParams(
            dimension_semantics=("parallel",),
            vmem_limit_bytes=48 * 1024 * 1024),
    )(scalars, x, wf, u_hat)
    return out, ld
